# baseline (speedup 1.0000x reference)
import jax
import jax.numpy as jnp
from jax.experimental import pallas as pl
from jax.experimental.pallas import tpu as pltpu

_NUM_CLASSES = 19
_DECAY = 0.99
_TAU = 0.05
_B, _C, _H, _W = 8, 3, 512, 512
_N = _H * _W
_K = _N // 4
_BH = 128
_NHB = _H // _BH


def _fused_body(cw_ref, cp_ref, bg_ref, cur_ref, prev_ref, ps_ref,
                imp_ref, wm_ref, ms_ref, ss_ref, cpo_ref, mask_ref, mi_ref,
                sms_ref, sss_ref):
    h = pl.program_id(1)

    @pl.when(h < _NHB)
    def _stream():
        cp = cp_ref[0]
        cpo_ref[0] = cp
        acc = cp[0] * cw_ref[0]
        mx = cp[0]
        for c in range(1, _NUM_CLASSES):
            acc = acc + cp[c] * cw_ref[c]
            mx = jnp.maximum(mx, cp[c])
        bg = bg_ref[0, 0]
        imp = jnp.where(mx > bg, acc, 0.0)
        dec = ps_ref[0, 0] * _DECAY
        wm = imp > dec + _TAU
        ss = jnp.where(wm, imp, dec)
        imp_ref[0, 0] = imp
        wm_ref[0, 0] = wm
        ss_ref[0, 0] = ss
        sss_ref[pl.ds(h * _BH, _BH), :] = ss
        for ch in range(_C):
            msc = jnp.where(wm, cur_ref[0, ch], prev_ref[0, ch])
            ms_ref[0, ch] = msc
            sms_ref[ch, pl.ds(h * _BH, _BH), :] = msc

    @pl.when(h == _NHB)
    def _topk():
        scores = sss_ref[...]
        bits = jax.lax.bitcast_convert_type(scores, jnp.int32)
        k = jnp.int32(_K)

        def vbody(_, carry):
            lo, hi = carry
            mid = lo + (hi - lo) // 2
            cnt = jnp.sum((bits >= mid).astype(jnp.int32))
            take = cnt >= k
            return jnp.where(take, mid, lo), jnp.where(take, hi, mid)

        t, _ = jax.lax.fori_loop(0, 30, vbody,
                                 (jnp.int32(0), jnp.int32(0x3F800000)))

        cnt_gt = jnp.sum((bits > t).astype(jnp.int32))
        s = k - cnt_gt

        eq = bits == t
        cnt_eq = jnp.sum(eq.astype(jnp.int32))
        row = jax.lax.broadcasted_iota(jnp.int32, (_H, _W), 0)
        col = jax.lax.broadcasted_iota(jnp.int32, (_H, _W), 1)
        idx = row * _W + col

        def _tie_search():
            def ibody(_, carry):
                lo, hi = carry
                mid = lo + (hi - lo) // 2
                cnt = jnp.sum((eq & (idx <= mid)).astype(jnp.int32))
                take = cnt >= s
                return jnp.where(take, lo, mid), jnp.where(take, mid, hi)

            return jax.lax.fori_loop(0, 18, ibody,
                                     (jnp.int32(-1), jnp.int32(_N - 1)))[1]

        p = jax.lax.cond(cnt_eq == s, lambda: jnp.int32(_N - 1), _tie_search)

        mask = (bits > t) | (eq & (idx <= p))
        mask_ref[0, 0] = mask
        mf = mask.astype(jnp.float32)
        for ch in range(_C):
            mi_ref[0, ch] = sms_ref[ch] * mf


def kernel(current_image, class_probs, background_prob, prev_memory,
           prev_scores, class_weights):
    f32 = jnp.float32
    cw = class_weights.reshape(_NUM_CLASSES).astype(f32)

    def _hmap(b, h):
        hc = jnp.minimum(h, _NHB - 1)
        return (b, 0, hc, 0)

    outs = pl.pallas_call(
        _fused_body,
        grid=(_B, _NHB + 1),
        in_specs=[
            pl.BlockSpec(memory_space=pltpu.SMEM),
            pl.BlockSpec((1, _NUM_CLASSES, _BH, _W), _hmap),
            pl.BlockSpec((1, 1, _BH, _W), _hmap),
            pl.BlockSpec((1, _C, _BH, _W), _hmap),
            pl.BlockSpec((1, _C, _BH, _W), _hmap),
            pl.BlockSpec((1, 1, _BH, _W), _hmap),
        ],
        out_specs=[
            pl.BlockSpec((1, 1, _BH, _W), _hmap),
            pl.BlockSpec((1, 1, _BH, _W), _hmap),
            pl.BlockSpec((1, _C, _BH, _W), _hmap),
            pl.BlockSpec((1, 1, _BH, _W), _hmap),
            pl.BlockSpec((1, _NUM_CLASSES, _BH, _W), _hmap),
            pl.BlockSpec((1, 1, _H, _W), lambda b, h: (b, 0, 0, 0)),
            pl.BlockSpec((1, _C, _H, _W), lambda b, h: (b, 0, 0, 0)),
        ],
        out_shape=[
            jax.ShapeDtypeStruct((_B, 1, _H, _W), f32),
            jax.ShapeDtypeStruct((_B, 1, _H, _W), jnp.bool_),
            jax.ShapeDtypeStruct((_B, _C, _H, _W), f32),
            jax.ShapeDtypeStruct((_B, 1, _H, _W), f32),
            jax.ShapeDtypeStruct((_B, _NUM_CLASSES, _H, _W), f32),
            jax.ShapeDtypeStruct((_B, 1, _H, _W), jnp.bool_),
            jax.ShapeDtypeStruct((_B, _C, _H, _W), f32),
        ],
        scratch_shapes=[
            pltpu.VMEM((_C, _H, _W), f32),
            pltpu.VMEM((_H, _W), f32),
        ],
    )(cw, class_probs, background_prob, current_image, prev_memory,
      prev_scores)
    imp, write_mask, mem_state, score_state, cp_out, output_mask, \
        memory_image = outs

    return (memory_image, mem_state, score_state, imp, write_mask,
            output_mask, cp_out)


# speedup vs baseline: 1.0202x; 1.0202x over previous
import jax
import jax.numpy as jnp
from jax.experimental import pallas as pl
from jax.experimental.pallas import tpu as pltpu

_NUM_CLASSES = 19
_DECAY = 0.99
_TAU = 0.05
_B, _C, _H, _W = 8, 3, 512, 512
_N = _H * _W
_K = _N // 4
_BH = 128
_NHB = _H // _BH


def _fused_body(cw_ref, cp_ref, bg_ref, cur_ref, prev_ref, ps_ref,
                imp_ref, wm_ref, ms_ref, ss_ref, cpo_ref, mask_ref, mi_ref,
                sms_ref, sss_ref):
    h = pl.program_id(1)

    @pl.when(h < _NHB)
    def _stream():
        cp = cp_ref[0]
        cpo_ref[0] = cp
        acc = cp[0] * cw_ref[0]
        mx = cp[0]
        for c in range(1, _NUM_CLASSES):
            acc = acc + cp[c] * cw_ref[c]
            mx = jnp.maximum(mx, cp[c])
        bg = bg_ref[0, 0]
        imp = jnp.where(mx > bg, acc, 0.0)
        dec = ps_ref[0, 0] * _DECAY
        wm = imp > dec + _TAU
        ss = jnp.where(wm, imp, dec)
        imp_ref[0, 0] = imp
        wm_ref[0, 0] = wm.astype(jnp.uint8)
        ss_ref[0, 0] = ss
        sss_ref[pl.ds(h * _BH, _BH), :] = ss
        for ch in range(_C):
            msc = jnp.where(wm, cur_ref[0, ch], prev_ref[0, ch])
            ms_ref[0, ch] = msc
            sms_ref[ch, pl.ds(h * _BH, _BH), :] = msc

    @pl.when(h == _NHB)
    def _topk():
        scores = sss_ref[...]
        bits = jax.lax.bitcast_convert_type(scores, jnp.int32)
        k = jnp.int32(_K)

        def vbody(_, carry):
            lo, hi = carry
            mid = lo + (hi - lo) // 2
            cnt = jnp.sum((bits >= mid).astype(jnp.int32))
            take = cnt >= k
            return jnp.where(take, mid, lo), jnp.where(take, hi, mid)

        t, _ = jax.lax.fori_loop(0, 30, vbody,
                                 (jnp.int32(0), jnp.int32(0x3F800000)))

        cnt_gt = jnp.sum((bits > t).astype(jnp.int32))
        s = k - cnt_gt

        eq = bits == t
        cnt_eq = jnp.sum(eq.astype(jnp.int32))
        row = jax.lax.broadcasted_iota(jnp.int32, (_H, _W), 0)
        col = jax.lax.broadcasted_iota(jnp.int32, (_H, _W), 1)
        idx = row * _W + col

        def _tie_search():
            def ibody(_, carry):
                lo, hi = carry
                mid = lo + (hi - lo) // 2
                cnt = jnp.sum((eq & (idx <= mid)).astype(jnp.int32))
                take = cnt >= s
                return jnp.where(take, lo, mid), jnp.where(take, mid, hi)

            return jax.lax.fori_loop(0, 18, ibody,
                                     (jnp.int32(-1), jnp.int32(_N - 1)))[1]

        p = jax.lax.cond(cnt_eq == s, lambda: jnp.int32(_N - 1), _tie_search)

        mask = (bits > t) | (eq & (idx <= p))
        mask_ref[0, 0] = mask.astype(jnp.uint8)
        mf = mask.astype(jnp.float32)
        for ch in range(_C):
            mi_ref[0, ch] = sms_ref[ch] * mf


def kernel(current_image, class_probs, background_prob, prev_memory,
           prev_scores, class_weights):
    f32 = jnp.float32
    cw = class_weights.reshape(_NUM_CLASSES).astype(f32)

    def _hmap(b, h):
        hc = jnp.minimum(h, _NHB - 1)
        return (b, 0, hc, 0)

    outs = pl.pallas_call(
        _fused_body,
        grid=(_B, _NHB + 1),
        in_specs=[
            pl.BlockSpec(memory_space=pltpu.SMEM),
            pl.BlockSpec((1, _NUM_CLASSES, _BH, _W), _hmap),
            pl.BlockSpec((1, 1, _BH, _W), _hmap),
            pl.BlockSpec((1, _C, _BH, _W), _hmap),
            pl.BlockSpec((1, _C, _BH, _W), _hmap),
            pl.BlockSpec((1, 1, _BH, _W), _hmap),
        ],
        out_specs=[
            pl.BlockSpec((1, 1, _BH, _W), _hmap),
            pl.BlockSpec((1, 1, _BH, _W), _hmap),
            pl.BlockSpec((1, _C, _BH, _W), _hmap),
            pl.BlockSpec((1, 1, _BH, _W), _hmap),
            pl.BlockSpec((1, _NUM_CLASSES, _BH, _W), _hmap),
            pl.BlockSpec((1, 1, _H, _W), lambda b, h: (b, 0, 0, 0)),
            pl.BlockSpec((1, _C, _H, _W), lambda b, h: (b, 0, 0, 0)),
        ],
        out_shape=[
            jax.ShapeDtypeStruct((_B, 1, _H, _W), f32),
            jax.ShapeDtypeStruct((_B, 1, _H, _W), jnp.uint8),
            jax.ShapeDtypeStruct((_B, _C, _H, _W), f32),
            jax.ShapeDtypeStruct((_B, 1, _H, _W), f32),
            jax.ShapeDtypeStruct((_B, _NUM_CLASSES, _H, _W), f32),
            jax.ShapeDtypeStruct((_B, 1, _H, _W), jnp.uint8),
            jax.ShapeDtypeStruct((_B, _C, _H, _W), f32),
        ],
        scratch_shapes=[
            pltpu.VMEM((_C, _H, _W), f32),
            pltpu.VMEM((_H, _W), f32),
        ],
    )(cw, class_probs, background_prob, current_image, prev_memory,
      prev_scores)
    imp, wm_u8, mem_state, score_state, cp_out, mask_u8, memory_image = outs

    write_mask = wm_u8.astype(jnp.bool_)
    output_mask = mask_u8.astype(jnp.bool_)
    return (memory_image, mem_state, score_state, imp, write_mask,
            output_mask, cp_out)


# speedup vs baseline: 1.0388x; 1.0182x over previous
import jax
import jax.numpy as jnp
from jax.experimental import pallas as pl
from jax.experimental.pallas import tpu as pltpu

_NUM_CLASSES = 19
_DECAY = 0.99
_TAU = 0.05
_B, _C, _H, _W = 8, 3, 512, 512
_N = _H * _W
_K = _N // 4
_BH = 128
_NHB = _H // _BH


def _fused_body(cw_ref, cp_ref, bg_ref, cur_ref, prev_ref, ps_ref,
                imp_ref, wm_ref, ms_ref, ss_ref, cpo_ref, mask_ref, mi_ref,
                sms_ref, sss_ref):
    h = pl.program_id(1)

    def _stream():
        cp = cp_ref[0]
        cpo_ref[0] = cp
        acc = cp[0] * cw_ref[0]
        mx = cp[0]
        for c in range(1, _NUM_CLASSES):
            acc = acc + cp[c] * cw_ref[c]
            mx = jnp.maximum(mx, cp[c])
        bg = bg_ref[0, 0]
        imp = jnp.where(mx > bg, acc, 0.0)
        dec = ps_ref[0, 0] * _DECAY
        wm = imp > dec + _TAU
        ss = jnp.where(wm, imp, dec)
        imp_ref[0, 0] = imp
        wm_ref[0, 0] = wm.astype(jnp.uint8)
        ss_ref[0, 0] = ss
        sss_ref[pl.ds(h * _BH, _BH), :] = ss
        for ch in range(_C):
            msc = jnp.where(wm, cur_ref[0, ch], prev_ref[0, ch])
            ms_ref[0, ch] = msc
            sms_ref[ch, pl.ds(h * _BH, _BH), :] = msc

    _stream()

    @pl.when(h == _NHB - 1)
    def _topk():
        scores = sss_ref[...]
        bits = jax.lax.bitcast_convert_type(scores, jnp.int32)
        k = jnp.int32(_K)

        def vbody(_, carry):
            lo, hi = carry
            mid = lo + (hi - lo) // 2
            cnt = jnp.sum((bits >= mid).astype(jnp.int32))
            take = cnt >= k
            return jnp.where(take, mid, lo), jnp.where(take, hi, mid)

        t, _ = jax.lax.fori_loop(0, 30, vbody,
                                 (jnp.int32(0), jnp.int32(0x3F800000)))

        cnt_gt = jnp.sum((bits > t).astype(jnp.int32))
        s = k - cnt_gt

        eq = bits == t
        cnt_eq = jnp.sum(eq.astype(jnp.int32))
        row = jax.lax.broadcasted_iota(jnp.int32, (_H, _W), 0)
        col = jax.lax.broadcasted_iota(jnp.int32, (_H, _W), 1)
        idx = row * _W + col

        def _tie_search():
            def ibody(_, carry):
                lo, hi = carry
                mid = lo + (hi - lo) // 2
                cnt = jnp.sum((eq & (idx <= mid)).astype(jnp.int32))
                take = cnt >= s
                return jnp.where(take, lo, mid), jnp.where(take, mid, hi)

            return jax.lax.fori_loop(0, 18, ibody,
                                     (jnp.int32(-1), jnp.int32(_N - 1)))[1]

        p = jax.lax.cond(cnt_eq == s, lambda: jnp.int32(_N - 1), _tie_search)

        mask = (bits > t) | (eq & (idx <= p))
        mask_ref[0, 0] = mask.astype(jnp.uint8)
        mf = mask.astype(jnp.float32)
        for ch in range(_C):
            mi_ref[0, ch] = sms_ref[ch] * mf


def kernel(current_image, class_probs, background_prob, prev_memory,
           prev_scores, class_weights):
    f32 = jnp.float32
    cw = class_weights.reshape(_NUM_CLASSES).astype(f32)

    def _hmap(b, h):
        return (b, 0, h, 0)

    outs = pl.pallas_call(
        _fused_body,
        grid=(_B, _NHB),
        in_specs=[
            pl.BlockSpec(memory_space=pltpu.SMEM),
            pl.BlockSpec((1, _NUM_CLASSES, _BH, _W), _hmap),
            pl.BlockSpec((1, 1, _BH, _W), _hmap),
            pl.BlockSpec((1, _C, _BH, _W), _hmap),
            pl.BlockSpec((1, _C, _BH, _W), _hmap),
            pl.BlockSpec((1, 1, _BH, _W), _hmap),
        ],
        out_specs=[
            pl.BlockSpec((1, 1, _BH, _W), _hmap),
            pl.BlockSpec((1, 1, _BH, _W), _hmap),
            pl.BlockSpec((1, _C, _BH, _W), _hmap),
            pl.BlockSpec((1, 1, _BH, _W), _hmap),
            pl.BlockSpec((1, _NUM_CLASSES, _BH, _W), _hmap),
            pl.BlockSpec((1, 1, _H, _W), lambda b, h: (b, 0, 0, 0)),
            pl.BlockSpec((1, _C, _H, _W), lambda b, h: (b, 0, 0, 0)),
        ],
        out_shape=[
            jax.ShapeDtypeStruct((_B, 1, _H, _W), f32),
            jax.ShapeDtypeStruct((_B, 1, _H, _W), jnp.uint8),
            jax.ShapeDtypeStruct((_B, _C, _H, _W), f32),
            jax.ShapeDtypeStruct((_B, 1, _H, _W), f32),
            jax.ShapeDtypeStruct((_B, _NUM_CLASSES, _H, _W), f32),
            jax.ShapeDtypeStruct((_B, 1, _H, _W), jnp.uint8),
            jax.ShapeDtypeStruct((_B, _C, _H, _W), f32),
        ],
        scratch_shapes=[
            pltpu.VMEM((_C, _H, _W), f32),
            pltpu.VMEM((_H, _W), f32),
        ],
    )(cw, class_probs, background_prob, current_image, prev_memory,
      prev_scores)
    imp, wm_u8, mem_state, score_state, cp_out, mask_u8, memory_image = outs

    write_mask = wm_u8.astype(jnp.bool_)
    output_mask = mask_u8.astype(jnp.bool_)
    return (memory_image, mem_state, score_state, imp, write_mask,
            output_mask, cp_out)
